# baseline (speedup 1.0000x reference)
import functools

import jax
import jax.numpy as jnp
from jax import lax
from jax.experimental import pallas as pl
from jax.experimental.pallas import tpu as pltpu
from jax.experimental.pallas import tpu_sc as plsc

EMBED = 32
CHUNK = 128


@functools.lru_cache(maxsize=None)
def _make_gather(B, D):
    info = plsc.get_sparse_core_info()
    NC, NS = info.num_cores, info.num_subcores
    NW = NC * NS
    b_per_w = B // NW
    n_chunks = b_per_w // CHUNK
    mesh = plsc.VectorSubcoreMesh(core_axis_name="c", subcore_axis_name="s")

    @functools.partial(
        pl.kernel,
        mesh=mesh,
        out_type=[
            jax.ShapeDtypeStruct((B, D), jnp.float32),
            jax.ShapeDtypeStruct((B, D), jnp.float32),
        ],
        scratch_types=[
            pltpu.VMEM((n_chunks, CHUNK), jnp.int32),
            pltpu.VMEM((n_chunks, CHUNK), jnp.int32),
            pltpu.VMEM((b_per_w, D), jnp.float32),
            pltpu.VMEM((b_per_w, D), jnp.float32),
            pltpu.SemaphoreType.DMA,
        ],
        compiler_params=pltpu.CompilerParams(use_tc_tiling_on_sc=False),
    )
    def gather_k(utab, itab, uidx, iidx, uout, iout, uidx_v, iidx_v, urows, irows, sem):
        wid = lax.axis_index("s") * NC + lax.axis_index("c")
        row0 = wid * n_chunks
        pltpu.sync_copy(uidx.at[pl.ds(row0, n_chunks)], uidx_v)
        pltpu.sync_copy(iidx.at[pl.ds(row0, n_chunks)], iidx_v)
        copies = []
        for j in range(n_chunks):
            copies.append(
                pltpu.async_copy(utab.at[uidx_v.at[j]], urows.at[pl.ds(j * CHUNK, CHUNK)], sem)
            )
            copies.append(
                pltpu.async_copy(itab.at[iidx_v.at[j]], irows.at[pl.ds(j * CHUNK, CHUNK)], sem)
            )
        for c in copies:
            c.wait()
        base = wid * b_per_w
        pltpu.sync_copy(urows, uout.at[pl.ds(base, b_per_w)])
        pltpu.sync_copy(irows, iout.at[pl.ds(base, b_per_w)])

    return gather_k


def _desc_body(desc, Wd, bd, out):
    out[...] = jnp.maximum(
        jnp.dot(desc[...], Wd[...], preferred_element_type=jnp.float32) + bd[...], 0.0
    )


def _desc_proj(desc, Wd, bd):
    B, K = desc.shape
    D = Wd.shape[1]
    BB = 2048
    return pl.pallas_call(
        _desc_body,
        grid=(B // BB,),
        in_specs=[
            pl.BlockSpec((BB, K), lambda i: (i, 0)),
            pl.BlockSpec(Wd.shape, lambda i: (0, 0)),
            pl.BlockSpec(bd.shape, lambda i: (0, 0)),
        ],
        out_specs=pl.BlockSpec((BB, D), lambda i: (i, 0)),
        out_shape=jax.ShapeDtypeStruct((B, D), jnp.float32),
    )(desc, Wd, bd)


def _mlp_body(uemb, iemb, dd, W1, b1, W2, b2, Wo, bo, out):
    W1v = W1[...]
    h = (
        jnp.dot(uemb[...], W1v[0:EMBED], preferred_element_type=jnp.float32)
        + jnp.dot(iemb[...], W1v[EMBED : 2 * EMBED], preferred_element_type=jnp.float32)
        + jnp.dot(dd[...], W1v[2 * EMBED :], preferred_element_type=jnp.float32)
        + b1[...]
    )
    h = jnp.maximum(h, 0.0)
    h2 = jnp.maximum(
        jnp.dot(h, W2[...], preferred_element_type=jnp.float32) + b2[...], 0.0
    )
    out[...] = jnp.sum(h2 * Wo[...], axis=1, keepdims=True) + bo[...]


def _mlp(uemb, iemb, dd, W1, b1, W2, b2, Wo, bo):
    B, D = uemb.shape
    BB = 4096

    def row_blk(shape):
        return pl.BlockSpec(shape, lambda i: (i, 0))

    def full_blk(shape):
        return pl.BlockSpec(shape, lambda i: (0, 0))

    return pl.pallas_call(
        _mlp_body,
        grid=(B // BB,),
        in_specs=[
            row_blk((BB, D)),
            row_blk((BB, D)),
            row_blk((BB, D)),
            full_blk(W1.shape),
            full_blk(b1.shape),
            full_blk(W2.shape),
            full_blk(b2.shape),
            full_blk(Wo.shape),
            full_blk(bo.shape),
        ],
        out_specs=row_blk((BB, 1)),
        out_shape=jax.ShapeDtypeStruct((B, 1), jnp.float32),
    )(uemb, iemb, dd, W1, b1, W2, b2, Wo, bo)


@jax.jit
def kernel(user_input, item_input, description_input, user_table, item_table,
           W_desc, b_desc, W1, b1, W2, b2, W_out, b_out):
    B = user_input.shape[0]
    uidx = user_input.reshape(B // CHUNK, CHUNK)
    iidx = item_input.reshape(B // CHUNK, CHUNK)
    uemb, iemb = _make_gather(B, EMBED)(user_table, item_table, uidx, iidx)
    dd = _desc_proj(description_input, W_desc, b_desc.reshape(1, -1))
    return _mlp(
        uemb, iemb, dd,
        W1, b1.reshape(1, -1),
        W2, b2.reshape(1, -1),
        W_out.reshape(1, -1), b_out.reshape(1, -1),
    )
